# Initial kernel scaffold; baseline (speedup 1.0000x reference)
import jax
import jax.numpy as jnp
from jax.experimental import pallas as pl


def kernel(users, items, dow_emb, time_emb, sex_emb, age_emb, month_emb, day_emb, W, b, item_table):
    raise NotImplementedError("write your pallas kernel here")


# speedup vs baseline: 1.4431x; 1.4431x over previous
import functools

import jax
import jax.numpy as jnp
from jax import lax
from jax.experimental import pallas as pl
from jax.experimental.pallas import tpu as pltpu
from jax.experimental.pallas import tpu_sc as plsc

_B = 4096
_L = 200
_ND = 8
_F = 32
_VOCABS = (7, 24, 2, 100, 12, 31)

_NC = 2
_NS = 16
_NW = _NC * _NS
_IPW = _B * _L // _NW
_CR = 8
_CI = _CR * _L
_NCH = _IPW // _CI
_GW = 100
_NG = _CI // _GW


def _user_proj_body(users_ref, dow_ref, time_ref, sex_ref, age_ref,
                    month_ref, day_ref, w_ref, b_ref, u_ref):
    tables = (dow_ref, time_ref, sex_ref, age_ref, month_ref, day_ref)
    u = jnp.broadcast_to(b_ref[...], (_B, _F))
    for k, (tbl, v) in enumerate(zip(tables, _VOCABS)):
        proj = jnp.dot(tbl[...], w_ref[k * _ND:(k + 1) * _ND, :],
                       preferred_element_type=jnp.float32)
        col = users_ref[:, k:k + 1]
        iota = lax.broadcasted_iota(jnp.int32, (_B, v), 1)
        onehot = (col == iota).astype(jnp.float32)
        u = u + jnp.dot(onehot, proj, preferred_element_type=jnp.float32)
    u_ref[...] = u


def _user_proj(users, dow, time, sex, age, month, day, w, b):
    return pl.pallas_call(
        _user_proj_body,
        out_shape=jax.ShapeDtypeStruct((_B, _F), jnp.float32),
    )(users, dow, time, sex, age, month, day, w, b.reshape(1, _F))


def _sc_dot(u, items2d, table):
    mesh = plsc.VectorSubcoreMesh(core_axis_name="c", subcore_axis_name="s")

    @functools.partial(
        pl.kernel,
        out_type=jax.ShapeDtypeStruct((_B * _L,), jnp.float32),
        mesh=mesh,
        compiler_params=pltpu.CompilerParams(needs_layout_passes=False,
                                             use_tc_tiling_on_sc=False),
        scratch_types=[
            pltpu.VMEM((_NG, _GW), jnp.int32),
            pltpu.VMEM((_CI + 8, _F), jnp.float32),
            pltpu.VMEM((_CR, _F), jnp.float32),
            pltpu.VMEM((_CI + 8,), jnp.float32),
            pltpu.SemaphoreType.DMA,
        ],
    )
    def k(u_hbm, items_hbm, table_hbm, out_hbm, idx_v, rows_v, u_v, out_v, sem):
        wid = lax.axis_index("s") * _NC + lax.axis_index("c")
        lanes = lax.iota(jnp.int32, 16)

        def chunk(ch, carry):
            ibase = wid * _IPW + ch * _CI
            pltpu.sync_copy(items_hbm.at[pl.ds(wid * (_IPW // _GW) + ch * _NG, _NG)],
                            idx_v)
            pltpu.sync_copy(u_hbm.at[pl.ds(wid * (_IPW // _L) + ch * _CR, _CR)], u_v)
            copies = [
                pltpu.async_copy(table_hbm.at[idx_v.at[g]],
                                 rows_v.at[pl.ds(g * _GW, _GW)], sem)
                for g in range(_NG)
            ]
            for cp in copies:
                cp.wait()

            def row(r, c2):
                rsplat = jnp.full((16,), r, jnp.int32)
                ubs = [plsc.load_gather(u_v, [rsplat, jnp.full((16,), f, jnp.int32)])
                       for f in range(_F)]

                def cch(c, c3):
                    base = r * _L + c * 16
                    ridx = base + lanes
                    acc = jnp.zeros((16,), jnp.float32)
                    for f in range(_F):
                        vals = plsc.load_gather(
                            rows_v, [ridx, jnp.full((16,), f, jnp.int32)])
                        acc = acc + ubs[f] * vals
                    out_v[pl.ds(base, 16)] = acc
                    return c3

                return lax.fori_loop(0, (_L + 15) // 16, cch, c2)

            lax.fori_loop(0, _CR, row, 0)
            pltpu.sync_copy(out_v.at[pl.ds(0, _CI)], out_hbm.at[pl.ds(ibase, _CI)])
            return carry

        lax.fori_loop(0, _NCH, chunk, 0)

    return k(u, items2d, table)


def kernel(users, items, dow_emb, time_emb, sex_emb, age_emb, month_emb,
           day_emb, W, b, item_table):
    users = users.astype(jnp.int32)
    items2d = items.astype(jnp.int32).reshape(_B * _L // _GW, _GW)
    u = _user_proj(users, dow_emb, time_emb, sex_emb, age_emb, month_emb,
                   day_emb, W, b)
    out = _sc_dot(u, items2d, item_table)
    return out.reshape(_B, _L)


# speedup vs baseline: 1.5652x; 1.0846x over previous
import functools

import jax
import jax.numpy as jnp
from jax import lax
from jax.experimental import pallas as pl
from jax.experimental.pallas import tpu as pltpu
from jax.experimental.pallas import tpu_sc as plsc

_B = 4096
_L = 200
_ND = 8
_F = 32
_VOCABS = (7, 24, 2, 100, 12, 31)

_NC = 2
_NS = 16
_NW = _NC * _NS
_RPW = _B // _NW
_IPW = _RPW * _L
_CR = 4
_CI = _CR * _L
_NCH = _RPW // _CR
_GA = 104
_GB = 96


def _user_proj_body(users_ref, dow_ref, time_ref, sex_ref, age_ref,
                    month_ref, day_ref, w_ref, b_ref, u_ref):
    tables = (dow_ref, time_ref, sex_ref, age_ref, month_ref, day_ref)
    u = jnp.broadcast_to(b_ref[...], (_B, _F))
    for k, (tbl, v) in enumerate(zip(tables, _VOCABS)):
        proj = jnp.dot(tbl[...], w_ref[k * _ND:(k + 1) * _ND, :],
                       preferred_element_type=jnp.float32)
        col = users_ref[:, k:k + 1]
        iota = lax.broadcasted_iota(jnp.int32, (_B, v), 1)
        onehot = (col == iota).astype(jnp.float32)
        u = u + jnp.dot(onehot, proj, preferred_element_type=jnp.float32)
    u_ref[...] = u


def _user_proj(users, dow, time, sex, age, month, day, w, b):
    return pl.pallas_call(
        _user_proj_body,
        out_shape=jax.ShapeDtypeStruct((_B, _F), jnp.float32),
    )(users, dow, time, sex, age, month, day, w, b.reshape(1, _F))


def _sc_dot(u, items, table):
    mesh = plsc.VectorSubcoreMesh(core_axis_name="c", subcore_axis_name="s")

    @functools.partial(
        pl.kernel,
        out_type=jax.ShapeDtypeStruct((_B * _L,), jnp.float32),
        mesh=mesh,
        compiler_params=pltpu.CompilerParams(needs_layout_passes=False,
                                             use_tc_tiling_on_sc=False),
        scratch_types=[
            pltpu.VMEM((_RPW, _GA), jnp.int32),
            pltpu.VMEM((_RPW, _GB), jnp.int32),
            pltpu.VMEM((_RPW, _F), jnp.float32),
            pltpu.VMEM((_CI + 8, _F), jnp.float32),
            pltpu.VMEM((_CI + 8, _F), jnp.float32),
            pltpu.VMEM((_CI + 8,), jnp.float32),
            pltpu.VMEM((_CI + 8,), jnp.float32),
            pltpu.SemaphoreType.DMA,
            pltpu.SemaphoreType.DMA,
            pltpu.SemaphoreType.DMA,
            pltpu.SemaphoreType.DMA,
        ],
    )
    def k(u_hbm, items_hbm, table_hbm, out_hbm,
          idx_a, idx_b, u_v, rows0, rows1, out0, out1, gs0, gs1, os0, os1):
        wid = lax.axis_index("s") * _NC + lax.axis_index("c")
        rbase = wid * _RPW
        lanes = lax.iota(jnp.int32, 16)

        pltpu.sync_copy(items_hbm.at[pl.ds(rbase, _RPW), pl.ds(0, _GA)], idx_a)
        pltpu.sync_copy(items_hbm.at[pl.ds(rbase, _RPW), pl.ds(_GA, _GB)], idx_b)
        pltpu.sync_copy(u_hbm.at[pl.ds(rbase, _RPW)], u_v)

        def fire_gathers(c, rows, gs):
            for r in range(_CR):
                row = c * _CR + r
                pltpu.async_copy(table_hbm.at[idx_a.at[row]],
                                 rows.at[pl.ds(r * _L, _GA)], gs)
                pltpu.async_copy(table_hbm.at[idx_b.at[row]],
                                 rows.at[pl.ds(r * _L + _GA, _GB)], gs)

        def wait_gathers(c, rows, gs):
            for r in range(_CR):
                pltpu.make_async_copy(table_hbm.at[idx_a.at[0]],
                                      rows.at[pl.ds(r * _L, _GA)], gs).wait()
                pltpu.make_async_copy(table_hbm.at[idx_b.at[0]],
                                      rows.at[pl.ds(r * _L + _GA, _GB)],
                                      gs).wait()

        def compute(c, rows, out_v):
            for r in range(_CR):
                row = c * _CR + r
                rsplat = jnp.full((16,), row, jnp.int32)
                u_lo = plsc.load_gather(u_v, [rsplat, lanes])
                u_hi = plsc.load_gather(u_v, [rsplat, lanes + 16])

                @plsc.parallel_loop(0, (_L + 15) // 16, 1)
                def cch(cc):
                    base = r * _L + cc * 16
                    ridx = base + lanes
                    acc = jnp.zeros((16,), jnp.float32)
                    for f in range(_F):
                        src = u_lo if f < 16 else u_hi
                        ub = src.at[jnp.full((16,), f % 16, jnp.int32)].get(
                            mode="promise_in_bounds")
                        vals = plsc.load_gather(
                            rows, [ridx, jnp.full((16,), f, jnp.int32)])
                        acc = acc + ub * vals
                    out_v[pl.ds(base, 16)] = acc

        def slot(c, rows, out_v, gs, os):
            wait_gathers(c, rows, gs)
            pl.when(c >= 2)(lambda: pltpu.make_async_copy(
                out_v.at[pl.ds(0, _CI)],
                out_hbm.at[pl.ds(wid * _IPW, _CI)], os).wait())
            compute(c, rows, out_v)
            obase = wid * _IPW + c * _CI
            pltpu.async_copy(out_v.at[pl.ds(0, _CI)],
                             out_hbm.at[pl.ds(obase, _CI)], os)
            pl.when(c + 2 < _NCH)(lambda: fire_gathers(c + 2, rows, gs))

        fire_gathers(0, rows0, gs0)
        fire_gathers(1, rows1, gs1)

        def pair(i, carry):
            slot(2 * i, rows0, out0, gs0, os0)
            slot(2 * i + 1, rows1, out1, gs1, os1)
            return carry

        lax.fori_loop(0, _NCH // 2, pair, 0)
        pltpu.make_async_copy(out0.at[pl.ds(0, _CI)],
                              out_hbm.at[pl.ds(wid * _IPW, _CI)], os0).wait()
        pltpu.make_async_copy(out1.at[pl.ds(0, _CI)],
                              out_hbm.at[pl.ds(wid * _IPW, _CI)], os1).wait()

    return k(u, items, table)


def kernel(users, items, dow_emb, time_emb, sex_emb, age_emb, month_emb,
           day_emb, W, b, item_table):
    users = users.astype(jnp.int32)
    items = items.astype(jnp.int32)
    u = _user_proj(users, dow_emb, time_emb, sex_emb, age_emb, month_emb,
                   day_emb, W, b)
    out = _sc_dot(u, items, item_table)
    return out.reshape(_B, _L)
